# Initial kernel scaffold; baseline (speedup 1.0000x reference)
import jax
import jax.numpy as jnp
from jax.experimental import pallas as pl


def kernel(x, table, pe):
    raise NotImplementedError("write your pallas kernel here")


# speedup vs baseline: 4.2838x; 4.2838x over previous
import functools

import jax
import jax.numpy as jnp
from jax import lax
from jax.experimental import pallas as pl
from jax.experimental.pallas import tpu as pltpu
from jax.experimental.pallas import tpu_sc as plsc

D = 128
L = 200
LANES = 16
GROUPS = D // LANES


@functools.lru_cache(maxsize=None)
def _build(B, V):
    NW = 32
    RPW = B // NW

    mesh = plsc.VectorSubcoreMesh(core_axis_name="c", subcore_axis_name="s")

    @functools.partial(
        pl.kernel,
        mesh=mesh,
        out_type=jax.ShapeDtypeStruct((B * L, D), jnp.float32),
        scratch_types=[
            pltpu.VMEM((L,), jnp.int32),
            pltpu.VMEM((L, D), jnp.float32),
            pltpu.VMEM((L, D), jnp.float32),
            pltpu.SemaphoreType.DMA,
        ],
    )
    def emb_pe(x_hbm, table_hbm, pe_hbm, out_hbm, idx_v, rows_v, pe_v, sem):
        wid = lax.axis_index("s") * 2 + lax.axis_index("c")
        pltpu.sync_copy(pe_hbm, pe_v)

        def body(i, carry):
            base = (wid * RPW + i) * L
            pltpu.sync_copy(x_hbm.at[pl.ds(base, L)], idx_v)
            pltpu.async_copy(table_hbm.at[idx_v], rows_v, sem).wait()

            def add_row(r, c):
                for g in range(GROUPS):
                    sl = pl.ds(g * LANES, LANES)
                    rows_v[r, sl] = rows_v[r, sl] + pe_v[r, sl]
                return c

            lax.fori_loop(0, L, add_row, 0)
            pltpu.sync_copy(rows_v, out_hbm.at[pl.ds(base, L)])
            return carry

        lax.fori_loop(0, RPW, body, 0)

    return emb_pe


def kernel(x, table, pe):
    B, Lx = x.shape
    xf = x.reshape(-1).astype(jnp.int32)
    pef = pe.reshape(Lx, D)
    out = _build(B, table.shape[0])(xf, table, pef)
    return out.reshape(B, Lx, D)


# speedup vs baseline: 7.5181x; 1.7550x over previous
import functools

import jax
import jax.numpy as jnp
from jax import lax
from jax.experimental import pallas as pl
from jax.experimental.pallas import tpu as pltpu
from jax.experimental.pallas import tpu_sc as plsc

D = 128
L = 200
LANES = 16
GROUPS = D // LANES


@functools.lru_cache(maxsize=None)
def _build(B, V):
    NW = 32
    RPW = B // NW

    mesh = plsc.VectorSubcoreMesh(core_axis_name="c", subcore_axis_name="s")

    @functools.partial(
        pl.kernel,
        mesh=mesh,
        out_type=jax.ShapeDtypeStruct((B * L, D), jnp.float32),
        scratch_types=[
            pltpu.VMEM((RPW * L,), jnp.int32),
            pltpu.VMEM((L, D), jnp.float32),
            pltpu.VMEM((L, D), jnp.float32),
            pltpu.VMEM((L, D), jnp.float32),
            pltpu.SemaphoreType.DMA,
            pltpu.SemaphoreType.DMA,
            pltpu.SemaphoreType.DMA,
            pltpu.SemaphoreType.DMA,
        ],
    )
    def emb_pe(x_hbm, table_hbm, pe_hbm, out_hbm,
               idx_v, pe_v, rows0, rows1, g0, g1, s0, s1):
        wid = lax.axis_index("s") * 2 + lax.axis_index("c")
        row0 = wid * RPW
        pltpu.sync_copy(pe_hbm, pe_v)
        pltpu.sync_copy(x_hbm.at[pl.ds(row0 * L, RPW * L)], idx_v)

        bufs = (rows0, rows1)
        gsems = (g0, g1)
        ssems = (s0, s1)

        def gather(j, b):
            return pltpu.make_async_copy(
                table_hbm.at[idx_v.at[pl.ds(j * L, L)]], bufs[b], gsems[b])

        def store(j, b):
            return pltpu.make_async_copy(
                bufs[b], out_hbm.at[pl.ds((row0 + j) * L, L)], ssems[b])

        gather(0, 0).start()

        def body(i2, carry):
            for b in (0, 1):
                o = 1 - b
                j = 2 * i2 + b

                @pl.when(j > 0)
                def _():
                    store(j - 1, o).wait()

                @pl.when(j < RPW - 1)
                def _():
                    gather(j + 1, o).start()

                gather(j, b).wait()

                def add_row(r, c):
                    for g in range(GROUPS):
                        sl = pl.ds(g * LANES, LANES)
                        bufs[b][r, sl] = bufs[b][r, sl] + pe_v[r, sl]
                    return c

                lax.fori_loop(0, L, add_row, 0)
                store(j, b).start()
            return carry

        lax.fori_loop(0, RPW // 2, body, 0)
        store(RPW - 1, 1).wait()

    return emb_pe


def kernel(x, table, pe):
    B, Lx = x.shape
    xi = x.reshape(-1).astype(jnp.int32)
    pef = pe.reshape(Lx, D)
    out = _build(B, table.shape[0])(xi, table, pef)
    return out.reshape(B, Lx, D)
